# baseline (speedup 1.0000x reference)
import functools

import jax
import jax.numpy as jnp
from jax import lax
from jax.experimental import pallas as pl
from jax.experimental.pallas import tpu as pltpu
from jax.experimental.pallas import tpu_sc as plsc

B = 16384
D = 200
L = 16
NC, NS = 2, 16
NW = NC * NS
RPW = B // NW
CPW = 128
NCHK = RPW // CPW
GPC = CPW // L


def _sc_body(xt_hbm, out_hbm, buf_a, buf_b, obuf, sem_a, sem_b):
    cid = lax.axis_index("c")
    sid = lax.axis_index("s")
    wid = sid * NC + cid
    base = wid * RPW

    bufs = (buf_a, buf_b)
    sems = (sem_a, sem_b)

    def start(ci):
        return pltpu.async_copy(
            xt_hbm.at[:, pl.ds(base + ci * CPW, CPW)], bufs[ci % 2], sems[ci % 2]
        )

    cps = [None] * NCHK
    for ci in range(2):
        cps[ci] = start(ci)

    lane = lax.broadcasted_iota(jnp.int32, (L,), 0)
    uzero = jnp.zeros((L,), jnp.uint32)
    uone = jnp.ones((L,), jnp.uint32)

    def make_grp(buf, obuf_base):
        def grp(g, carry):
            r0 = g * L
            acc = [uzero, uzero, uzero, uzero]
            for c in range(D):
                v = plsc.bitcast(buf[c, pl.ds(r0, L)], jnp.uint32)
                acc[c % 4] = acc[c % 4] + jnp.minimum(v << 1, uone)
            cnt = plsc.bitcast((acc[0] + acc[1]) + (acc[2] + acc[3]), jnp.int32)
            idx_c = jnp.maximum(cnt - 1, 0)
            obuf[pl.ds(obuf_base + r0, L)] = plsc.load_gather(
                buf, [idx_c, r0 + lane]
            )
            return carry

        return grp

    for ci in range(NCHK):
        cps[ci].wait()
        lax.fori_loop(0, GPC, make_grp(bufs[ci % 2], ci * CPW), 0)
        if ci + 2 < NCHK:
            cps[ci + 2] = start(ci + 2)

    pltpu.sync_copy(obuf, out_hbm.at[pl.ds(base, RPW)])


@jax.jit
def kernel(x):
    mesh = plsc.VectorSubcoreMesh(core_axis_name="c", subcore_axis_name="s")
    f = functools.partial(
        pl.kernel,
        out_type=jax.ShapeDtypeStruct((B,), jnp.float32),
        mesh=mesh,
        scratch_types=[
            pltpu.VMEM((D, CPW), jnp.float32),
            pltpu.VMEM((D, CPW), jnp.float32),
            pltpu.VMEM((RPW,), jnp.float32),
            pltpu.SemaphoreType.DMA,
            pltpu.SemaphoreType.DMA,
        ],
        compiler_params=pltpu.CompilerParams(
            needs_layout_passes=False,
            skip_device_barrier=True,
        ),
    )(_sc_body)
    return f(x.T)


# speedup vs baseline: 1.7193x; 1.7193x over previous
import functools

import jax
import jax.numpy as jnp
from jax import lax
from jax.experimental import pallas as pl
from jax.experimental.pallas import tpu as pltpu
from jax.experimental.pallas import tpu_sc as plsc

B = 16384
D = 200
L = 16
NC, NS = 2, 16
NW = NC * NS
RPW = B // NW
NSPLIT = 2
CPW = RPW // NSPLIT
GPC = CPW // L


def _sc_body(xt_hbm, out_hbm, buf_a, buf_b, obuf, sem_a, sem_b):
    cid = lax.axis_index("c")
    sid = lax.axis_index("s")
    wid = sid * NC + cid
    base = wid * RPW

    bufs = (buf_a, buf_b)
    cps = [
        pltpu.async_copy(
            xt_hbm.at[:, pl.ds(base + ci * CPW, CPW)], bufs[ci], (sem_a, sem_b)[ci]
        )
        for ci in range(NSPLIT)
    ]

    lane = lax.broadcasted_iota(jnp.int32, (L,), 0)
    uzero = jnp.zeros((L,), jnp.uint32)
    uone = jnp.ones((L,), jnp.uint32)

    def make_grp(buf, obuf_base):
        def grp(g, carry):
            r0 = g * L
            acc = [uzero, uzero, uzero, uzero]
            for c in range(D):
                v = plsc.bitcast(buf[c, pl.ds(r0, L)], jnp.uint32)
                acc[c % 4] = acc[c % 4] + jnp.minimum(v << 1, uone)
            cnt = plsc.bitcast((acc[0] + acc[1]) + (acc[2] + acc[3]), jnp.int32)
            idx_c = jnp.maximum(cnt - 1, 0)
            obuf[pl.ds(obuf_base + r0, L)] = plsc.load_gather(
                buf, [idx_c, r0 + lane]
            )
            return carry

        return grp

    for ci in range(NSPLIT):
        cps[ci].wait()
        lax.fori_loop(0, GPC, make_grp(bufs[ci], ci * CPW), 0)

    pltpu.sync_copy(obuf, out_hbm.at[pl.ds(base, RPW)])


@jax.jit
def kernel(x):
    mesh = plsc.VectorSubcoreMesh(core_axis_name="c", subcore_axis_name="s")
    f = functools.partial(
        pl.kernel,
        out_type=jax.ShapeDtypeStruct((B,), jnp.float32),
        mesh=mesh,
        scratch_types=[
            pltpu.VMEM((D, CPW), jnp.float32),
            pltpu.VMEM((D, CPW), jnp.float32),
            pltpu.VMEM((RPW,), jnp.float32),
            pltpu.SemaphoreType.DMA,
            pltpu.SemaphoreType.DMA,
        ],
        compiler_params=pltpu.CompilerParams(needs_layout_passes=False),
    )(_sc_body)
    return f(x.T)
